# baseline (speedup 1.0000x reference)
import numpy as np

import jax
import jax.numpy as jnp
from jax.experimental import pallas as pl
from jax.experimental.pallas import tpu as pltpu

N = 10000
IN_F = 128
OUT_F = 64
G = 5
K = 3
NB = G + K

TP = 1000
TM = 200
P1 = N // TP
P2 = N // TM

_GRID = tuple(
    (np.arange(-K, G + K + 1, dtype=np.float32) * np.float32(2.0 / G)
     - np.float32(1.0)).tolist()
)


def _b_spline_bases(x):
    g = [np.float32(v) for v in _GRID]
    inv_h = np.float32(1.0) / (g[1] - g[0])
    u = (x - g[0]) * inv_h
    cf = jnp.floor(u)
    t = u - cf
    t2 = t * t
    t3 = t2 * t
    c16 = np.float32(1.0 / 6.0)
    c12 = np.float32(0.5)
    s0 = t3 * c16
    s1 = ((-c12 * t + c12) * t + c12) * t + c16
    s2 = (c12 * t - np.float32(1.0)) * t2 + np.float32(4.0 / 6.0)
    s3 = ((-c16 * t + c12) * t - c12) * t + c16
    seg = (s0, s1, s2, s3)
    bases = []
    for j in range(NB):
        b = jnp.zeros_like(x)
        for m in range(4):
            b = b + jnp.where(cf == np.float32(j + m), seg[m], np.float32(0.0))
        bases.append(b)
    return bases


def _kan(x, bwt, swt):
    y = jnp.dot(jax.nn.silu(x), bwt, preferred_element_type=jnp.float32)
    for j, b in enumerate(_b_spline_bases(x)):
        y = y + jnp.dot(b, swt[j], preferred_element_type=jnp.float32)
    return y


def _mega_kernel(adj_ref, gn_ref, feat_ref, feat_a_ref,
                 bw1t_ref, sw1t_ref, bw2t_ref, sw2t_ref, dw_ref, db_ref,
                 z_ref, h_ref, ret_ref, reta_ref,
                 zz0_s, emb_s, h0_s):
    i = pl.program_id(0)

    @pl.when(i < P1)
    def _phase1():
        bwt = bw1t_ref[...]
        swt = sw1t_ref[...]
        zz0_s[pl.ds(i * TP, TP), :OUT_F] = _kan(feat_ref[...], bwt, swt)
        zz0_s[pl.ds(i * TP, TP), OUT_F:] = _kan(feat_a_ref[...], bwt, swt)

    @pl.when((i >= P1) & (i < P1 + P2))
    def _phase2():
        r = i - P1
        s = jnp.dot(adj_ref[...], zz0_s[...],
                    preferred_element_type=jnp.float32)
        z = s[:, :OUT_F]
        z_ref[...] = z
        emb_s[pl.ds(r * TM, TM), :] = jnp.where(
            s > 0, s, jnp.exp(jnp.minimum(s, 0.0)) - 1.0)
        h0_s[pl.ds(r * TM, TM), :] = _kan(z, bw2t_ref[...], sw2t_ref[...])

    @pl.when(i >= P1 + P2)
    def _phase3():
        r = i - (P1 + P2)
        h_ref[...] = jnp.dot(adj_ref[...], h0_s[...],
                             preferred_element_type=jnp.float32)
        gn = gn_ref[...]
        rr = jnp.dot(gn, emb_s[...], preferred_element_type=jnp.float32)
        row_sum = jnp.sum(gn, axis=1, keepdims=True)

        ge = rr[:, :OUT_F] / row_sum
        ga = rr[:, OUT_F:] / row_sum
        nrm_e = jnp.maximum(
            jnp.sqrt(jnp.sum(ge * ge, axis=1, keepdims=True)), 1e-12)
        nrm_a = jnp.maximum(
            jnp.sqrt(jnp.sum(ga * ga, axis=1, keepdims=True)), 1e-12)
        g = jax.nn.sigmoid(ge / nrm_e)
        g_a = jax.nn.sigmoid(ga / nrm_a)

        emb_tile = emb_s[pl.ds(r * TM, TM), :]
        e = emb_tile[:, :OUT_F]
        ea = emb_tile[:, OUT_F:]
        dw = dw_ref[...]
        b = db_ref[0, 0]
        p = jnp.dot(e, dw, preferred_element_type=jnp.float32)
        pa = jnp.dot(ea, dw, preferred_element_type=jnp.float32)
        ret_ref[:, 0:1] = jnp.sum(p * g, axis=1, keepdims=True) + b
        ret_ref[:, 1:2] = jnp.sum(pa * g, axis=1, keepdims=True) + b
        reta_ref[:, 0:1] = jnp.sum(pa * g_a, axis=1, keepdims=True) + b
        reta_ref[:, 1:2] = jnp.sum(p * g_a, axis=1, keepdims=True) + b


def _whole(shape):
    return pl.BlockSpec(shape, lambda i: tuple(0 for _ in shape))


def kernel(feat, feat_a, adj, graph_neigh, base_w1, spline_w1,
           base_w2, spline_w2, disc_W, disc_b):
    f32 = jnp.float32
    bw1t = base_w1.T
    sw1t = jnp.transpose(spline_w1, (2, 1, 0))
    bw2t = base_w2.T
    sw2t = jnp.transpose(spline_w2, (2, 1, 0))
    db = disc_b.reshape(1, 1)

    z, h, ret, ret_a = pl.pallas_call(
        _mega_kernel,
        grid=(P1 + 2 * P2,),
        in_specs=[
            pl.BlockSpec(
                (TM, N),
                lambda i: (jnp.where(i >= P1 + P2, i - (P1 + P2),
                                     jnp.clip(i - P1, 0, P2 - 1)), 0)),
            pl.BlockSpec((TM, N), lambda i: (jnp.maximum(i - (P1 + P2), 0), 0)),
            pl.BlockSpec((TP, IN_F), lambda i: (jnp.clip(i, 0, P1 - 1), 0)),
            pl.BlockSpec((TP, IN_F), lambda i: (jnp.clip(i, 0, P1 - 1), 0)),
            _whole((IN_F, OUT_F)),
            _whole((NB, IN_F, OUT_F)),
            _whole((OUT_F, IN_F)),
            _whole((NB, OUT_F, IN_F)),
            _whole((OUT_F, OUT_F)),
            _whole((1, 1)),
        ],
        out_specs=[
            pl.BlockSpec((TM, OUT_F),
                         lambda i: (jnp.clip(i - P1, 0, P2 - 1), 0)),
            pl.BlockSpec((TM, IN_F),
                         lambda i: (jnp.maximum(i - (P1 + P2), 0), 0)),
            pl.BlockSpec((TM, 2), lambda i: (jnp.maximum(i - (P1 + P2), 0), 0)),
            pl.BlockSpec((TM, 2), lambda i: (jnp.maximum(i - (P1 + P2), 0), 0)),
        ],
        out_shape=[
            jax.ShapeDtypeStruct((N, OUT_F), f32),
            jax.ShapeDtypeStruct((N, IN_F), f32),
            jax.ShapeDtypeStruct((N, 2), f32),
            jax.ShapeDtypeStruct((N, 2), f32),
        ],
        scratch_shapes=[
            pltpu.VMEM((N, 2 * OUT_F), f32),
            pltpu.VMEM((N, 2 * OUT_F), f32),
            pltpu.VMEM((N, IN_F), f32),
        ],
        compiler_params=pltpu.CompilerParams(
            dimension_semantics=("arbitrary",)),
    )(adj, graph_neigh, feat, feat_a, bw1t, sw1t, bw2t, sw2t, disc_W, db)

    return (z, h, ret, ret_a)
